# baseline (speedup 1.0000x reference)
import functools

import jax
import jax.numpy as jnp
from jax import lax
from jax.experimental import pallas as pl
from jax.experimental.pallas import tpu as pltpu
from jax.experimental.pallas import tpu_sc as plsc

N = 10000
NP = 10240
E = 160000
NS = 16
NC = 2
EPT = E // NS
KB = 112
NBLK = 91
EPAD = NBLK * KB - EPT
ROWS_PT = NP // NS
BM = 1000


def _sc_agg(C):
    P = C // NC
    mesh = plsc.VectorSubcoreMesh(core_axis_name="c", subcore_axis_name="s")

    @functools.partial(
        pl.kernel,
        out_type=jax.ShapeDtypeStruct((C * NP, 128), jnp.float32),
        mesh=mesh,
        scratch_types=[
            pltpu.VMEM((NBLK, KB), jnp.int32),
            pltpu.VMEM((NBLK, KB), jnp.int32),
            pltpu.VMEM((KB, 128), jnp.float32),
            pltpu.VMEM((KB, 128), jnp.float32),
            pltpu.VMEM_SHARED((NP, 128), jnp.float32),
            pltpu.SemaphoreType.DMA,
            pltpu.SemaphoreType.DMA,
        ],
        compiler_params=pltpu.CompilerParams(use_tc_tiling_on_sc=False),
    )
    def k(y, src, dst, out, src_v, dst_v, buf_a, buf_b, acc, sem_a, sem_b):
        c = lax.axis_index("c")
        s = lax.axis_index("s")
        pltpu.sync_copy(src.at[s], src_v)
        pltpu.sync_copy(dst.at[s], dst_v)
        for p in range(P):
            base = (c * P + p) * NP
            pltpu.sync_copy(y.at[pl.ds(base + s * ROWS_PT, ROWS_PT)],
                            acc.at[pl.ds(s * ROWS_PT, ROWS_PT)])
            plsc.subcore_barrier()

            ytab = y.at[pl.ds(base, NP)]
            pltpu.async_copy(ytab.at[src_v.at[0]], buf_a, sem_a)

            def body(i, carry):
                j0 = 2 * i
                pltpu.make_async_copy(
                    ytab.at[src_v.at[0]], buf_a, sem_a).wait()
                pltpu.async_copy(ytab.at[src_v.at[j0 + 1]], buf_b, sem_b)
                pltpu.sync_copy(buf_a, acc.at[dst_v.at[j0]], add=True)
                pltpu.make_async_copy(
                    ytab.at[src_v.at[0]], buf_b, sem_b).wait()
                pltpu.async_copy(ytab.at[src_v.at[j0 + 2]], buf_a, sem_a)
                pltpu.sync_copy(buf_b, acc.at[dst_v.at[j0 + 1]], add=True)
                return carry

            lax.fori_loop(0, (NBLK - 1) // 2, body, 0)
            pltpu.make_async_copy(
                ytab.at[src_v.at[0]], buf_a, sem_a).wait()
            pltpu.sync_copy(buf_a, acc.at[dst_v.at[NBLK - 1]], add=True)
            plsc.subcore_barrier()

            pltpu.sync_copy(acc.at[pl.ds(s * ROWS_PT, ROWS_PT)],
                            out.at[pl.ds(base + s * ROWS_PT, ROWS_PT)])
            plsc.subcore_barrier()

    return k


def _sc_deg():
    mesh = plsc.VectorSubcoreMesh(core_axis_name="c", subcore_axis_name="s")

    @functools.partial(
        pl.kernel,
        out_type=jax.ShapeDtypeStruct((NP, 16), jnp.float32),
        mesh=mesh,
        scratch_types=[
            pltpu.VMEM((NBLK, KB), jnp.int32),
            pltpu.VMEM((KB, 16), jnp.float32),
            pltpu.VMEM((ROWS_PT, 16), jnp.float32),
            pltpu.VMEM_SHARED((NP, 16), jnp.float32),
        ],
        compiler_params=pltpu.CompilerParams(use_tc_tiling_on_sc=False),
    )
    def k(dst, out, dst_v, ones_v, rows_v, acc):
        c = lax.axis_index("c")
        s = lax.axis_index("s")

        @pl.when(c == 0)
        def _():
            pltpu.sync_copy(dst.at[s], dst_v)

            def fill_ones(i, carry):
                ones_v[i, :] = jnp.full((16,), 1.0, jnp.float32)
                return carry

            lax.fori_loop(0, KB, fill_ones, 0)

            def fill_rows(i, carry):
                rows_v[i, :] = jnp.full((16,), 1.0, jnp.float32)
                return carry

            lax.fori_loop(0, ROWS_PT, fill_rows, 0)
            pltpu.sync_copy(rows_v, acc.at[pl.ds(s * ROWS_PT, ROWS_PT)])
            plsc.subcore_barrier()

            def body(j, carry):
                pltpu.sync_copy(ones_v, acc.at[dst_v.at[j]], add=True)
                return carry

            lax.fori_loop(0, NBLK, body, 0)
            plsc.subcore_barrier()
            pltpu.sync_copy(acc.at[pl.ds(s * ROWS_PT, ROWS_PT)], rows_v)
            pltpu.sync_copy(rows_v, out.at[pl.ds(s * ROWS_PT, ROWS_PT)])

    return k


def _leaky(x):
    return jnp.where(x > 0, x, 0.01 * x)


def _tc_matmul_first(x, w, deg):
    k_in, d_out = w.shape
    ck, cn = k_in // 128, d_out // 128
    grid = (N // BM, cn, ck)

    def body(x_ref, w_ref, deg_ref, out_ref):
        kk = pl.program_id(2)
        contrib = jnp.dot(x_ref[...], w_ref[...],
                          preferred_element_type=jnp.float32)

        @pl.when(kk == 0)
        def _():
            out_ref[0] = jnp.zeros_like(out_ref[0])

        out_ref[0] += contrib

        @pl.when(kk == ck - 1)
        def _():
            out_ref[0] = out_ref[0] * lax.rsqrt(deg_ref[:, 0:1])

    return pl.pallas_call(
        body,
        grid=grid,
        in_specs=[
            pl.BlockSpec((BM, 128), lambda m, n, k: (m, k)),
            pl.BlockSpec((128, 128), lambda m, n, k: (k, n)),
            pl.BlockSpec((BM, 16), lambda m, n, k: (m, 0)),
        ],
        out_specs=pl.BlockSpec((1, BM, 128), lambda m, n, k: (n, m, 0)),
        out_shape=jax.ShapeDtypeStruct((cn, NP, 128), jnp.float32),
    )(x, w, deg)


def _tc_matmul(agg, w, b_prev, deg):
    k_in, d_out = w.shape
    ck, cn = k_in // 128, d_out // 128
    grid = (N // BM, cn, ck)

    def body(agg_ref, w_ref, b_ref, deg_ref, out_ref):
        kk = pl.program_id(2)
        dinv = lax.rsqrt(deg_ref[:, 0:1])
        h = _leaky(dinv * agg_ref[0] + b_ref[0, 0])
        contrib = jnp.dot(h, w_ref[...], preferred_element_type=jnp.float32)

        @pl.when(kk == 0)
        def _():
            out_ref[0] = jnp.zeros_like(out_ref[0])

        out_ref[0] += contrib

        @pl.when(kk == ck - 1)
        def _():
            out_ref[0] = out_ref[0] * dinv

    return pl.pallas_call(
        body,
        grid=grid,
        in_specs=[
            pl.BlockSpec((1, BM, 128), lambda m, n, k: (k, m, 0)),
            pl.BlockSpec((128, 128), lambda m, n, k: (k, n)),
            pl.BlockSpec((1, 1, 128), lambda m, n, k: (k, 0, 0)),
            pl.BlockSpec((BM, 16), lambda m, n, k: (m, 0)),
        ],
        out_specs=pl.BlockSpec((1, BM, 128), lambda m, n, k: (n, m, 0)),
        out_shape=jax.ShapeDtypeStruct((cn, NP, 128), jnp.float32),
    )(agg, w, b_prev.reshape(ck, 1, 128), deg)


def _tc_epilogue(agg, b, deg, d_out):
    cn = d_out // 128
    grid = (N // BM, cn)

    def body(agg_ref, b_ref, deg_ref, out_ref):
        out_ref[...] = lax.rsqrt(deg_ref[:, 0:1]) * agg_ref[0] + b_ref[0, 0]

    return pl.pallas_call(
        body,
        grid=grid,
        in_specs=[
            pl.BlockSpec((1, BM, 128), lambda m, n: (n, m, 0)),
            pl.BlockSpec((1, 1, 128), lambda m, n: (n, 0, 0)),
            pl.BlockSpec((BM, 16), lambda m, n: (m, 0)),
        ],
        out_specs=pl.BlockSpec((BM, 128), lambda m, n: (m, n)),
        out_shape=jax.ShapeDtypeStruct((N, d_out), jnp.float32),
    )(agg, b.reshape(cn, 1, 128), deg)


_deg_kernel = _sc_deg()
_agg4 = _sc_agg(4)
_agg2 = _sc_agg(2)


def kernel(x, edge_index, W_enc, b_enc, W_h0, b_h0, W_h1, b_h1, W_h2, b_h2,
           W_dec, b_dec):
    src = jnp.pad(edge_index[0].reshape(NS, EPT),
                  ((0, 0), (0, EPAD))).reshape(NS, NBLK, KB)
    dst = jnp.pad(edge_index[1].reshape(NS, EPT), ((0, 0), (0, EPAD)),
                  constant_values=NP - 8).reshape(NS, NBLK, KB)

    deg = _deg_kernel(dst)

    y = _tc_matmul_first(x, W_enc, deg)
    agg = _agg4(y.reshape(4 * NP, 128), src, dst)
    y = _tc_matmul(agg.reshape(4, NP, 128), W_h0, b_enc, deg)
    agg = _agg4(y.reshape(4 * NP, 128), src, dst)
    y = _tc_matmul(agg.reshape(4, NP, 128), W_h1, b_h0, deg)
    agg = _agg4(y.reshape(4 * NP, 128), src, dst)
    y = _tc_matmul(agg.reshape(4, NP, 128), W_h2, b_h1, deg)
    agg = _agg4(y.reshape(4 * NP, 128), src, dst)
    y = _tc_matmul(agg.reshape(4, NP, 128), W_dec, b_h2, deg)
    agg = _agg2(y.reshape(2 * NP, 128), src, dst)
    return _tc_epilogue(agg.reshape(2, NP, 128), b_dec, deg, 256)


# speedup vs baseline: 1.4146x; 1.4146x over previous
import functools

import jax
import jax.numpy as jnp
from jax import lax
from jax.experimental import pallas as pl
from jax.experimental.pallas import tpu as pltpu
from jax.experimental.pallas import tpu_sc as plsc

N = 10000
NP = 10240
E = 160000
NS = 16
NC = 2
EPT = E // NS
KB = 80
NBLK = 125
EPAD = NBLK * KB - EPT
ROWS_PT = NP // NS
BM = 1000


def _sc_agg(C):
    P = C // NC
    mesh = plsc.VectorSubcoreMesh(core_axis_name="c", subcore_axis_name="s")

    @functools.partial(
        pl.kernel,
        out_type=jax.ShapeDtypeStruct((C * NP, 128), jnp.float32),
        mesh=mesh,
        scratch_types=[
            pltpu.VMEM((NBLK, KB), jnp.int32),
            pltpu.VMEM((NBLK, KB), jnp.int32),
            pltpu.VMEM((KB, 128), jnp.float32),
            pltpu.VMEM((KB, 128), jnp.float32),
            pltpu.VMEM_SHARED((NP, 128), jnp.float32),
            pltpu.SemaphoreType.DMA,
            pltpu.SemaphoreType.DMA,
            pltpu.SemaphoreType.DMA,
            pltpu.SemaphoreType.DMA,
        ],
        compiler_params=pltpu.CompilerParams(use_tc_tiling_on_sc=False),
    )
    def k(y, src, dst, out, src_v, dst_v, buf_a, buf_b, acc,
          sem_a, sem_b, sem_sa, sem_sb):
        c = lax.axis_index("c")
        s = lax.axis_index("s")
        pltpu.sync_copy(src.at[s], src_v)
        pltpu.sync_copy(dst.at[s], dst_v)
        for p in range(P):
            base = (c * P + p) * NP
            pltpu.sync_copy(y.at[pl.ds(base + s * ROWS_PT, ROWS_PT)],
                            acc.at[pl.ds(s * ROWS_PT, ROWS_PT)])
            plsc.subcore_barrier()

            ytab = y.at[pl.ds(base, NP)]
            pltpu.async_copy(ytab.at[src_v.at[0]], buf_a, sem_a)
            pltpu.async_copy(ytab.at[src_v.at[1]], buf_b, sem_b)

            def body(i, carry):
                j0 = 2 * i
                pltpu.make_async_copy(
                    ytab.at[src_v.at[0]], buf_a, sem_a).wait()
                pltpu.async_copy(buf_a, acc.at[dst_v.at[j0]], sem_sa,
                                 add=True)
                pltpu.make_async_copy(
                    ytab.at[src_v.at[0]], buf_b, sem_b).wait()
                pltpu.async_copy(buf_b, acc.at[dst_v.at[j0 + 1]], sem_sb,
                                 add=True)
                pltpu.make_async_copy(buf_a, acc.at[dst_v.at[0]],
                                      sem_sa).wait()
                pltpu.async_copy(ytab.at[src_v.at[j0 + 2]], buf_a, sem_a)
                pltpu.make_async_copy(buf_b, acc.at[dst_v.at[0]],
                                      sem_sb).wait()

                @pl.when(j0 + 3 < NBLK)
                def _():
                    pltpu.async_copy(ytab.at[src_v.at[j0 + 3]], buf_b, sem_b)

                return carry

            lax.fori_loop(0, (NBLK - 1) // 2, body, 0)
            pltpu.make_async_copy(
                ytab.at[src_v.at[0]], buf_a, sem_a).wait()
            pltpu.sync_copy(buf_a, acc.at[dst_v.at[NBLK - 1]], add=True)
            plsc.subcore_barrier()

            pltpu.sync_copy(acc.at[pl.ds(s * ROWS_PT, ROWS_PT)],
                            out.at[pl.ds(base + s * ROWS_PT, ROWS_PT)])
            plsc.subcore_barrier()

    return k


def _sc_deg():
    mesh = plsc.VectorSubcoreMesh(core_axis_name="c", subcore_axis_name="s")

    @functools.partial(
        pl.kernel,
        out_type=jax.ShapeDtypeStruct((NP, 16), jnp.float32),
        mesh=mesh,
        scratch_types=[
            pltpu.VMEM((NBLK, KB), jnp.int32),
            pltpu.VMEM((KB, 16), jnp.float32),
            pltpu.VMEM((ROWS_PT, 16), jnp.float32),
            pltpu.VMEM_SHARED((NP, 16), jnp.float32),
        ],
        compiler_params=pltpu.CompilerParams(use_tc_tiling_on_sc=False),
    )
    def k(dst, out, dst_v, ones_v, rows_v, acc):
        c = lax.axis_index("c")
        s = lax.axis_index("s")

        @pl.when(c == 0)
        def _():
            pltpu.sync_copy(dst.at[s], dst_v)

            def fill_ones(i, carry):
                ones_v[i, :] = jnp.full((16,), 1.0, jnp.float32)
                return carry

            lax.fori_loop(0, KB, fill_ones, 0)

            def fill_rows(i, carry):
                rows_v[i, :] = jnp.full((16,), 1.0, jnp.float32)
                return carry

            lax.fori_loop(0, ROWS_PT, fill_rows, 0)
            pltpu.sync_copy(rows_v, acc.at[pl.ds(s * ROWS_PT, ROWS_PT)])
            plsc.subcore_barrier()

            def body(j, carry):
                pltpu.sync_copy(ones_v, acc.at[dst_v.at[j]], add=True)
                return carry

            lax.fori_loop(0, NBLK, body, 0)
            plsc.subcore_barrier()
            pltpu.sync_copy(acc.at[pl.ds(s * ROWS_PT, ROWS_PT)], rows_v)
            pltpu.sync_copy(rows_v, out.at[pl.ds(s * ROWS_PT, ROWS_PT)])

    return k


def _leaky(x):
    return jnp.where(x > 0, x, 0.01 * x)


def _tc_matmul_first(x, w, deg):
    k_in, d_out = w.shape
    ck, cn = k_in // 128, d_out // 128
    grid = (N // BM, cn, ck)

    def body(x_ref, w_ref, deg_ref, out_ref):
        kk = pl.program_id(2)
        contrib = jnp.dot(x_ref[...], w_ref[...],
                          preferred_element_type=jnp.float32)

        @pl.when(kk == 0)
        def _():
            out_ref[0] = jnp.zeros_like(out_ref[0])

        out_ref[0] += contrib

        @pl.when(kk == ck - 1)
        def _():
            out_ref[0] = out_ref[0] * lax.rsqrt(deg_ref[:, 0:1])

    return pl.pallas_call(
        body,
        grid=grid,
        in_specs=[
            pl.BlockSpec((BM, 128), lambda m, n, k: (m, k)),
            pl.BlockSpec((128, 128), lambda m, n, k: (k, n)),
            pl.BlockSpec((BM, 16), lambda m, n, k: (m, 0)),
        ],
        out_specs=pl.BlockSpec((1, BM, 128), lambda m, n, k: (n, m, 0)),
        out_shape=jax.ShapeDtypeStruct((cn, NP, 128), jnp.float32),
    )(x, w, deg)


def _tc_matmul(agg, w, b_prev, deg):
    k_in, d_out = w.shape
    ck, cn = k_in // 128, d_out // 128
    grid = (N // BM, cn, ck)

    def body(agg_ref, w_ref, b_ref, deg_ref, out_ref):
        kk = pl.program_id(2)
        dinv = lax.rsqrt(deg_ref[:, 0:1])
        h = _leaky(dinv * agg_ref[0] + b_ref[0, 0])
        contrib = jnp.dot(h, w_ref[...], preferred_element_type=jnp.float32)

        @pl.when(kk == 0)
        def _():
            out_ref[0] = jnp.zeros_like(out_ref[0])

        out_ref[0] += contrib

        @pl.when(kk == ck - 1)
        def _():
            out_ref[0] = out_ref[0] * dinv

    return pl.pallas_call(
        body,
        grid=grid,
        in_specs=[
            pl.BlockSpec((1, BM, 128), lambda m, n, k: (k, m, 0)),
            pl.BlockSpec((128, 128), lambda m, n, k: (k, n)),
            pl.BlockSpec((1, 1, 128), lambda m, n, k: (k, 0, 0)),
            pl.BlockSpec((BM, 16), lambda m, n, k: (m, 0)),
        ],
        out_specs=pl.BlockSpec((1, BM, 128), lambda m, n, k: (n, m, 0)),
        out_shape=jax.ShapeDtypeStruct((cn, NP, 128), jnp.float32),
    )(agg, w, b_prev.reshape(ck, 1, 128), deg)


def _tc_epilogue(agg, b, deg, d_out):
    cn = d_out // 128
    grid = (N // BM, cn)

    def body(agg_ref, b_ref, deg_ref, out_ref):
        out_ref[...] = lax.rsqrt(deg_ref[:, 0:1]) * agg_ref[0] + b_ref[0, 0]

    return pl.pallas_call(
        body,
        grid=grid,
        in_specs=[
            pl.BlockSpec((1, BM, 128), lambda m, n: (n, m, 0)),
            pl.BlockSpec((1, 1, 128), lambda m, n: (n, 0, 0)),
            pl.BlockSpec((BM, 16), lambda m, n: (m, 0)),
        ],
        out_specs=pl.BlockSpec((BM, 128), lambda m, n: (m, n)),
        out_shape=jax.ShapeDtypeStruct((N, d_out), jnp.float32),
    )(agg, b.reshape(cn, 1, 128), deg)


_deg_kernel = _sc_deg()
_agg4 = _sc_agg(4)
_agg2 = _sc_agg(2)


def kernel(x, edge_index, W_enc, b_enc, W_h0, b_h0, W_h1, b_h1, W_h2, b_h2,
           W_dec, b_dec):
    src = edge_index[0].reshape(NS, NBLK, KB)
    dst = edge_index[1].reshape(NS, NBLK, KB)

    deg = _deg_kernel(dst)

    y = _tc_matmul_first(x, W_enc, deg)
    agg = _agg4(y.reshape(4 * NP, 128), src, dst)
    y = _tc_matmul(agg.reshape(4, NP, 128), W_h0, b_enc, deg)
    agg = _agg4(y.reshape(4 * NP, 128), src, dst)
    y = _tc_matmul(agg.reshape(4, NP, 128), W_h1, b_h0, deg)
    agg = _agg4(y.reshape(4 * NP, 128), src, dst)
    y = _tc_matmul(agg.reshape(4, NP, 128), W_h2, b_h1, deg)
    agg = _agg4(y.reshape(4 * NP, 128), src, dst)
    y = _tc_matmul(agg.reshape(4, NP, 128), W_dec, b_h2, deg)
    agg = _agg2(y.reshape(2 * NP, 128), src, dst)
    return _tc_epilogue(agg.reshape(2, NP, 128), b_dec, deg, 256)


# speedup vs baseline: 1.8293x; 1.2932x over previous
import functools

import jax
import jax.numpy as jnp
from jax import lax
from jax.experimental import pallas as pl
from jax.experimental.pallas import tpu as pltpu
from jax.experimental.pallas import tpu_sc as plsc

N = 10000
NP = 10240
E = 160000
NS = 16
NC = 2
EPT = E // NS
KB = 80
NBLK = 125
EPAD = NBLK * KB - EPT
ROWS_PT = NP // NS
BM = 1000


def _sc_agg(C):
    P = C // NC
    mesh = plsc.VectorSubcoreMesh(core_axis_name="c", subcore_axis_name="s")

    @functools.partial(
        pl.kernel,
        out_type=jax.ShapeDtypeStruct((C * NP, 128), jnp.float32),
        mesh=mesh,
        scratch_types=[
            pltpu.VMEM((NBLK, KB), jnp.int32),
            pltpu.VMEM((NBLK, KB), jnp.int32),
            pltpu.VMEM((KB, 128), jnp.float32),
            pltpu.VMEM((KB, 128), jnp.float32),
            pltpu.VMEM_SHARED((NP, 128), jnp.float32),
            pltpu.SemaphoreType.DMA,
            pltpu.SemaphoreType.DMA,
            pltpu.SemaphoreType.DMA,
            pltpu.SemaphoreType.DMA,
        ],
        compiler_params=pltpu.CompilerParams(use_tc_tiling_on_sc=False),
    )
    def k(y, src, dst, out, src_v, dst_v, buf_a, buf_b, acc,
          sem_a, sem_b, sem_sa, sem_sb):
        c = lax.axis_index("c")
        s = lax.axis_index("s")
        pltpu.sync_copy(src.at[s], src_v)
        pltpu.sync_copy(dst.at[s], dst_v)
        for p in range(P):
            base = (c * P + p) * NP
            pltpu.sync_copy(y.at[pl.ds(base + s * ROWS_PT, ROWS_PT)],
                            acc.at[pl.ds(s * ROWS_PT, ROWS_PT)])
            plsc.subcore_barrier()

            ytab = y.at[pl.ds(base, NP)]
            pltpu.async_copy(ytab.at[src_v.at[0]], buf_a, sem_a)
            pltpu.async_copy(ytab.at[src_v.at[1]], buf_b, sem_b)

            def body(i, carry):
                j0 = 2 * i
                pltpu.make_async_copy(
                    ytab.at[src_v.at[0]], buf_a, sem_a).wait()
                pltpu.async_copy(buf_a, acc.at[dst_v.at[j0]], sem_sa,
                                 add=True)
                pltpu.make_async_copy(
                    ytab.at[src_v.at[0]], buf_b, sem_b).wait()
                pltpu.async_copy(buf_b, acc.at[dst_v.at[j0 + 1]], sem_sb,
                                 add=True)
                pltpu.make_async_copy(buf_a, acc.at[dst_v.at[0]],
                                      sem_sa).wait()
                pltpu.async_copy(ytab.at[src_v.at[j0 + 2]], buf_a, sem_a)
                pltpu.make_async_copy(buf_b, acc.at[dst_v.at[0]],
                                      sem_sb).wait()

                @pl.when(j0 + 3 < NBLK)
                def _():
                    pltpu.async_copy(ytab.at[src_v.at[j0 + 3]], buf_b, sem_b)

                return carry

            lax.fori_loop(0, (NBLK - 1) // 2, body, 0)
            pltpu.make_async_copy(
                ytab.at[src_v.at[0]], buf_a, sem_a).wait()
            pltpu.sync_copy(buf_a, acc.at[dst_v.at[NBLK - 1]], add=True)
            plsc.subcore_barrier()

            pltpu.sync_copy(acc.at[pl.ds(s * ROWS_PT, ROWS_PT)],
                            out.at[pl.ds(base + s * ROWS_PT, ROWS_PT)])
            plsc.subcore_barrier()

    return k


def _sc_deg():
    mesh = plsc.VectorSubcoreMesh(core_axis_name="c", subcore_axis_name="s")

    @functools.partial(
        pl.kernel,
        out_type=jax.ShapeDtypeStruct((NP, 16), jnp.float32),
        mesh=mesh,
        scratch_types=[
            pltpu.VMEM((NBLK, KB), jnp.int32),
            pltpu.VMEM((KB, 16), jnp.float32),
            pltpu.VMEM((ROWS_PT, 16), jnp.float32),
            pltpu.VMEM_SHARED((NP, 16), jnp.float32),
        ],
        compiler_params=pltpu.CompilerParams(use_tc_tiling_on_sc=False),
    )
    def k(dst, out, dst_v, ones_v, rows_v, acc):
        c = lax.axis_index("c")
        s = lax.axis_index("s")

        @pl.when(c == 0)
        def _():
            pltpu.sync_copy(dst.at[s], dst_v)

            def fill_ones(i, carry):
                ones_v[i, :] = jnp.full((16,), 1.0, jnp.float32)
                return carry

            lax.fori_loop(0, KB, fill_ones, 0)

            def fill_rows(i, carry):
                rows_v[i, :] = jnp.full((16,), 1.0, jnp.float32)
                return carry

            lax.fori_loop(0, ROWS_PT, fill_rows, 0)
            pltpu.sync_copy(rows_v, acc.at[pl.ds(s * ROWS_PT, ROWS_PT)])
            plsc.subcore_barrier()

            def body(j, carry):
                pltpu.sync_copy(ones_v, acc.at[dst_v.at[j]], add=True)
                return carry

            lax.fori_loop(0, NBLK, body, 0)
            plsc.subcore_barrier()
            pltpu.sync_copy(acc.at[pl.ds(s * ROWS_PT, ROWS_PT)], rows_v)
            pltpu.sync_copy(rows_v, out.at[pl.ds(s * ROWS_PT, ROWS_PT)])

    return k


def _leaky(x):
    return jnp.where(x > 0, x, 0.01 * x)


def _tc_matmul_first(x, w, deg):
    k_in, d_out = w.shape
    cn = d_out // 128

    def body(x_ref, w_ref, deg_ref, out_ref):
        y = jnp.dot(x_ref[...], w_ref[...],
                    preferred_element_type=jnp.float32)
        y = y * lax.rsqrt(deg_ref[:, 0:1])
        for c in range(cn):
            out_ref[c] = y[:, c * 128:(c + 1) * 128]

    return pl.pallas_call(
        body,
        grid=(N // BM,),
        in_specs=[
            pl.BlockSpec((BM, k_in), lambda m: (m, 0)),
            pl.BlockSpec((k_in, d_out), lambda m: (0, 0)),
            pl.BlockSpec((BM, 16), lambda m: (m, 0)),
        ],
        out_specs=pl.BlockSpec((cn, BM, 128), lambda m: (0, m, 0)),
        out_shape=jax.ShapeDtypeStruct((cn, NP, 128), jnp.float32),
    )(x, w, deg)


def _tc_matmul(agg, w, b_prev, deg):
    k_in, d_out = w.shape
    ck, cn = k_in // 128, d_out // 128

    def body(agg_ref, w_ref, b_ref, deg_ref, out_ref):
        dinv = lax.rsqrt(deg_ref[:, 0:1])
        acc = jnp.zeros((BM, d_out), jnp.float32)
        for c in range(ck):
            h = _leaky(dinv * agg_ref[c] + b_ref[0, c * 128:(c + 1) * 128])
            acc += jnp.dot(h, w_ref[c * 128:(c + 1) * 128, :],
                           preferred_element_type=jnp.float32)
        acc = acc * dinv
        for c in range(cn):
            out_ref[c] = acc[:, c * 128:(c + 1) * 128]

    return pl.pallas_call(
        body,
        grid=(N // BM,),
        in_specs=[
            pl.BlockSpec((ck, BM, 128), lambda m: (0, m, 0)),
            pl.BlockSpec((k_in, d_out), lambda m: (0, 0)),
            pl.BlockSpec((1, k_in), lambda m: (0, 0)),
            pl.BlockSpec((BM, 16), lambda m: (m, 0)),
        ],
        out_specs=pl.BlockSpec((cn, BM, 128), lambda m: (0, m, 0)),
        out_shape=jax.ShapeDtypeStruct((cn, NP, 128), jnp.float32),
    )(agg, w, b_prev.reshape(1, k_in), deg)


def _tc_epilogue(agg, b, deg, d_out):
    cn = d_out // 128
    grid = (N // BM, cn)

    def body(agg_ref, b_ref, deg_ref, out_ref):
        out_ref[...] = lax.rsqrt(deg_ref[:, 0:1]) * agg_ref[0] + b_ref[0, 0]

    return pl.pallas_call(
        body,
        grid=grid,
        in_specs=[
            pl.BlockSpec((1, BM, 128), lambda m, n: (n, m, 0)),
            pl.BlockSpec((1, 1, 128), lambda m, n: (n, 0, 0)),
            pl.BlockSpec((BM, 16), lambda m, n: (m, 0)),
        ],
        out_specs=pl.BlockSpec((BM, 128), lambda m, n: (m, n)),
        out_shape=jax.ShapeDtypeStruct((N, d_out), jnp.float32),
    )(agg, b.reshape(cn, 1, 128), deg)


_deg_kernel = _sc_deg()
_agg4 = _sc_agg(4)
_agg2 = _sc_agg(2)


def kernel(x, edge_index, W_enc, b_enc, W_h0, b_h0, W_h1, b_h1, W_h2, b_h2,
           W_dec, b_dec):
    src = edge_index[0].reshape(NS, NBLK, KB)
    dst = edge_index[1].reshape(NS, NBLK, KB)

    deg = _deg_kernel(dst)

    y = _tc_matmul_first(x, W_enc, deg)
    agg = _agg4(y.reshape(4 * NP, 128), src, dst)
    y = _tc_matmul(agg.reshape(4, NP, 128), W_h0, b_enc, deg)
    agg = _agg4(y.reshape(4 * NP, 128), src, dst)
    y = _tc_matmul(agg.reshape(4, NP, 128), W_h1, b_h0, deg)
    agg = _agg4(y.reshape(4 * NP, 128), src, dst)
    y = _tc_matmul(agg.reshape(4, NP, 128), W_h2, b_h1, deg)
    agg = _agg4(y.reshape(4 * NP, 128), src, dst)
    y = _tc_matmul(agg.reshape(4, NP, 128), W_dec, b_h2, deg)
    agg = _agg2(y.reshape(2 * NP, 128), src, dst)
    return _tc_epilogue(agg.reshape(2, NP, 128), b_dec, deg, 256)


# speedup vs baseline: 2.5947x; 1.4184x over previous
import functools

import jax
import jax.numpy as jnp
from jax import lax
from jax.experimental import pallas as pl
from jax.experimental.pallas import tpu as pltpu
from jax.experimental.pallas import tpu_sc as plsc

N = 10000
NP = 10240
E = 160000
NS = 16
NC = 2
EPT = E // NS
KB = 80
NBLK = 125
EPAD = NBLK * KB - EPT
ROWS_PT = NP // NS
BM = 1000


def _sc_agg(C):
    P = C // NC
    gather_only = C == 4
    scatter_only = C == 2
    mesh = plsc.VectorSubcoreMesh(core_axis_name="c", subcore_axis_name="s")

    @functools.partial(
        pl.kernel,
        out_type=jax.ShapeDtypeStruct((C * NP, 128), jnp.float32),
        mesh=mesh,
        scratch_types=[
            pltpu.VMEM((NBLK, KB), jnp.int32),
            pltpu.VMEM((NBLK, KB), jnp.int32),
            pltpu.VMEM((KB, 128), jnp.float32),
            pltpu.VMEM((KB, 128), jnp.float32),
            pltpu.VMEM_SHARED((NP, 128), jnp.float32),
            pltpu.SemaphoreType.DMA,
            pltpu.SemaphoreType.DMA,
            pltpu.SemaphoreType.DMA,
            pltpu.SemaphoreType.DMA,
        ],
        compiler_params=pltpu.CompilerParams(use_tc_tiling_on_sc=False),
    )
    def k(y, src, dst, out, src_v, dst_v, buf_a, buf_b, acc,
          sem_a, sem_b, sem_sa, sem_sb):
        c = lax.axis_index("c")
        s = lax.axis_index("s")
        pltpu.sync_copy(src.at[s], src_v)
        pltpu.sync_copy(dst.at[s], dst_v)
        for p in range(P):
            base = (c * P + p) * NP
            pltpu.sync_copy(y.at[pl.ds(base + s * ROWS_PT, ROWS_PT)],
                            acc.at[pl.ds(s * ROWS_PT, ROWS_PT)])
            plsc.subcore_barrier()

            ytab = y.at[pl.ds(base, NP)]
            pltpu.async_copy(ytab.at[src_v.at[0]], buf_a, sem_a)
            pltpu.async_copy(ytab.at[src_v.at[1]], buf_b, sem_b)

            def body(i, carry):
                j0 = 2 * i
                if gather_only:
                    pltpu.make_async_copy(
                        ytab.at[src_v.at[0]], buf_a, sem_a).wait()
                    pltpu.async_copy(ytab.at[src_v.at[j0 + 2]], buf_a, sem_a)
                    pltpu.make_async_copy(
                        ytab.at[src_v.at[0]], buf_b, sem_b).wait()

                    @pl.when(j0 + 3 < NBLK)
                    def _():
                        pltpu.async_copy(ytab.at[src_v.at[j0 + 3]], buf_b,
                                         sem_b)
                else:
                    pltpu.async_copy(buf_a, acc.at[dst_v.at[j0]], sem_sa,
                                     add=True)
                    pltpu.async_copy(buf_b, acc.at[dst_v.at[j0 + 1]], sem_sb,
                                     add=True)
                    pltpu.make_async_copy(buf_a, acc.at[dst_v.at[0]],
                                          sem_sa).wait()
                    pltpu.make_async_copy(buf_b, acc.at[dst_v.at[0]],
                                          sem_sb).wait()
                return carry

            lax.fori_loop(0, (NBLK - 1) // 2, body, 0)
            pltpu.make_async_copy(
                ytab.at[src_v.at[0]], buf_a, sem_a).wait()
            pltpu.sync_copy(buf_a, acc.at[dst_v.at[NBLK - 1]], add=True)
            plsc.subcore_barrier()

            pltpu.sync_copy(acc.at[pl.ds(s * ROWS_PT, ROWS_PT)],
                            out.at[pl.ds(base + s * ROWS_PT, ROWS_PT)])
            plsc.subcore_barrier()

    return k


def _sc_deg():
    mesh = plsc.VectorSubcoreMesh(core_axis_name="c", subcore_axis_name="s")

    @functools.partial(
        pl.kernel,
        out_type=jax.ShapeDtypeStruct((NP, 16), jnp.float32),
        mesh=mesh,
        scratch_types=[
            pltpu.VMEM((NBLK, KB), jnp.int32),
            pltpu.VMEM((KB, 16), jnp.float32),
            pltpu.VMEM((ROWS_PT, 16), jnp.float32),
            pltpu.VMEM_SHARED((NP, 16), jnp.float32),
        ],
        compiler_params=pltpu.CompilerParams(use_tc_tiling_on_sc=False),
    )
    def k(dst, out, dst_v, ones_v, rows_v, acc):
        c = lax.axis_index("c")
        s = lax.axis_index("s")

        @pl.when(c == 0)
        def _():
            pltpu.sync_copy(dst.at[s], dst_v)

            def fill_ones(i, carry):
                ones_v[i, :] = jnp.full((16,), 1.0, jnp.float32)
                return carry

            lax.fori_loop(0, KB, fill_ones, 0)

            def fill_rows(i, carry):
                rows_v[i, :] = jnp.full((16,), 1.0, jnp.float32)
                return carry

            lax.fori_loop(0, ROWS_PT, fill_rows, 0)
            pltpu.sync_copy(rows_v, acc.at[pl.ds(s * ROWS_PT, ROWS_PT)])
            plsc.subcore_barrier()

            def body(j, carry):
                pltpu.sync_copy(ones_v, acc.at[dst_v.at[j]], add=True)
                return carry

            lax.fori_loop(0, NBLK, body, 0)
            plsc.subcore_barrier()
            pltpu.sync_copy(acc.at[pl.ds(s * ROWS_PT, ROWS_PT)], rows_v)
            pltpu.sync_copy(rows_v, out.at[pl.ds(s * ROWS_PT, ROWS_PT)])

    return k


def _leaky(x):
    return jnp.where(x > 0, x, 0.01 * x)


def _tc_matmul_first(x, w, deg):
    k_in, d_out = w.shape
    cn = d_out // 128

    def body(x_ref, w_ref, deg_ref, out_ref):
        y = jnp.dot(x_ref[...], w_ref[...],
                    preferred_element_type=jnp.float32)
        y = y * lax.rsqrt(deg_ref[:, 0:1])
        for c in range(cn):
            out_ref[c] = y[:, c * 128:(c + 1) * 128]

    return pl.pallas_call(
        body,
        grid=(N // BM,),
        in_specs=[
            pl.BlockSpec((BM, k_in), lambda m: (m, 0)),
            pl.BlockSpec((k_in, d_out), lambda m: (0, 0)),
            pl.BlockSpec((BM, 16), lambda m: (m, 0)),
        ],
        out_specs=pl.BlockSpec((cn, BM, 128), lambda m: (0, m, 0)),
        out_shape=jax.ShapeDtypeStruct((cn, NP, 128), jnp.float32),
    )(x, w, deg)


def _tc_matmul(agg, w, b_prev, deg):
    k_in, d_out = w.shape
    ck, cn = k_in // 128, d_out // 128

    def body(agg_ref, w_ref, b_ref, deg_ref, out_ref):
        dinv = lax.rsqrt(deg_ref[:, 0:1])
        acc = jnp.zeros((BM, d_out), jnp.float32)
        for c in range(ck):
            h = _leaky(dinv * agg_ref[c] + b_ref[0, c * 128:(c + 1) * 128])
            acc += jnp.dot(h, w_ref[c * 128:(c + 1) * 128, :],
                           preferred_element_type=jnp.float32)
        acc = acc * dinv
        for c in range(cn):
            out_ref[c] = acc[:, c * 128:(c + 1) * 128]

    return pl.pallas_call(
        body,
        grid=(N // BM,),
        in_specs=[
            pl.BlockSpec((ck, BM, 128), lambda m: (0, m, 0)),
            pl.BlockSpec((k_in, d_out), lambda m: (0, 0)),
            pl.BlockSpec((1, k_in), lambda m: (0, 0)),
            pl.BlockSpec((BM, 16), lambda m: (m, 0)),
        ],
        out_specs=pl.BlockSpec((cn, BM, 128), lambda m: (0, m, 0)),
        out_shape=jax.ShapeDtypeStruct((cn, NP, 128), jnp.float32),
    )(agg, w, b_prev.reshape(1, k_in), deg)


def _tc_epilogue(agg, b, deg, d_out):
    cn = d_out // 128
    grid = (N // BM, cn)

    def body(agg_ref, b_ref, deg_ref, out_ref):
        out_ref[...] = lax.rsqrt(deg_ref[:, 0:1]) * agg_ref[0] + b_ref[0, 0]

    return pl.pallas_call(
        body,
        grid=grid,
        in_specs=[
            pl.BlockSpec((1, BM, 128), lambda m, n: (n, m, 0)),
            pl.BlockSpec((1, 1, 128), lambda m, n: (n, 0, 0)),
            pl.BlockSpec((BM, 16), lambda m, n: (m, 0)),
        ],
        out_specs=pl.BlockSpec((BM, 128), lambda m, n: (m, n)),
        out_shape=jax.ShapeDtypeStruct((N, d_out), jnp.float32),
    )(agg, b.reshape(cn, 1, 128), deg)


_deg_kernel = _sc_deg()
_agg4 = _sc_agg(4)
_agg2 = _sc_agg(2)


def kernel(x, edge_index, W_enc, b_enc, W_h0, b_h0, W_h1, b_h1, W_h2, b_h2,
           W_dec, b_dec):
    src = edge_index[0].reshape(NS, NBLK, KB)
    dst = edge_index[1].reshape(NS, NBLK, KB)

    deg = _deg_kernel(dst)

    y = _tc_matmul_first(x, W_enc, deg)
    agg = _agg4(y.reshape(4 * NP, 128), src, dst)
    y = _tc_matmul(agg.reshape(4, NP, 128), W_h0, b_enc, deg)
    agg = _agg4(y.reshape(4 * NP, 128), src, dst)
    y = _tc_matmul(agg.reshape(4, NP, 128), W_h1, b_h0, deg)
    agg = _agg4(y.reshape(4 * NP, 128), src, dst)
    y = _tc_matmul(agg.reshape(4, NP, 128), W_h2, b_h1, deg)
    agg = _agg4(y.reshape(4 * NP, 128), src, dst)
    y = _tc_matmul(agg.reshape(4, NP, 128), W_dec, b_h2, deg)
    agg = _agg2(y.reshape(2 * NP, 128), src, dst)
    return _tc_epilogue(agg.reshape(2, NP, 128), b_dec, deg, 256)


# speedup vs baseline: 2.7122x; 1.0453x over previous
import functools

import jax
import jax.numpy as jnp
from jax import lax
from jax.experimental import pallas as pl
from jax.experimental.pallas import tpu as pltpu
from jax.experimental.pallas import tpu_sc as plsc

N = 10000
NP = 10008
E = 160000
NS = 16
NC = 2
EPT = E // NS
KB = 80
NBLK = 125
ROWS_A = 632
ROWS_B = NP - 15 * ROWS_A
BM = 1000


def _sc_agg(C):
    P = C // NC
    mesh = plsc.VectorSubcoreMesh(core_axis_name="c", subcore_axis_name="s")

    @functools.partial(
        pl.kernel,
        out_type=jax.ShapeDtypeStruct((C * NP, 128), jnp.float32),
        mesh=mesh,
        scratch_types=[
            pltpu.VMEM((NBLK, KB), jnp.int32),
            pltpu.VMEM((NBLK, KB), jnp.int32),
            pltpu.VMEM((KB, 128), jnp.float32),
            pltpu.VMEM((KB, 128), jnp.float32),
            pltpu.VMEM((KB, 128), jnp.float32),
            pltpu.VMEM_SHARED((NP, 128), jnp.float32),
            pltpu.SemaphoreType.DMA,
            pltpu.SemaphoreType.DMA,
            pltpu.SemaphoreType.DMA,
            pltpu.SemaphoreType.DMA,
            pltpu.SemaphoreType.DMA,
            pltpu.SemaphoreType.DMA,
        ],
        compiler_params=pltpu.CompilerParams(use_tc_tiling_on_sc=False),
    )
    def k(y, src, dst, out, src_v, dst_v, buf_a, buf_b, buf_c, acc,
          sem_a, sem_b, sem_c, sem_sa, sem_sb, sem_sc):
        c = lax.axis_index("c")
        s = lax.axis_index("s")
        pltpu.sync_copy(src.at[s], src_v)
        pltpu.sync_copy(dst.at[s], dst_v)
        bufs = (buf_a, buf_b, buf_c)
        gsems = (sem_a, sem_b, sem_c)
        ssems = (sem_sa, sem_sb, sem_sc)

        def gwait(r):
            pltpu.make_async_copy(
                y.at[src_v.at[0]], bufs[r % 3], gsems[r % 3]).wait()

        def swait(r):
            pltpu.make_async_copy(
                bufs[r % 3], acc.at[dst_v.at[0]], ssems[r % 3]).wait()

        for p in range(P):
            base = (c * P + p) * NP
            ytab = y.at[pl.ds(base, NP)]
            pltpu.async_copy(ytab.at[src_v.at[0]], buf_a, sem_a)
            pltpu.async_copy(ytab.at[src_v.at[1]], buf_b, sem_b)

            @pl.when(s < 15)
            def _():
                pltpu.sync_copy(y.at[pl.ds(base + s * ROWS_A, ROWS_A)],
                                acc.at[pl.ds(s * ROWS_A, ROWS_A)])

            @pl.when(s == 15)
            def _():
                pltpu.sync_copy(y.at[pl.ds(base + 15 * ROWS_A, ROWS_B)],
                                acc.at[pl.ds(15 * ROWS_A, ROWS_B)])

            plsc.subcore_barrier()

            def stage(j, r, fire_j, loop_j0):
                gwait(r)
                pltpu.async_copy(bufs[r % 3], acc.at[dst_v.at[j]],
                                 ssems[r % 3], add=True)

                @pl.when(loop_j0 + r >= 1)
                def _():
                    swait(r + 2)

                @pl.when(fire_j < NBLK)
                def _():
                    pltpu.async_copy(ytab.at[src_v.at[fire_j]],
                                     bufs[(r + 2) % 3], gsems[(r + 2) % 3])

            def body(i, carry):
                j0 = 3 * i
                stage(j0, 0, j0 + 2, j0)

                @pl.when(j0 + 1 < NBLK)
                def _():
                    stage(j0 + 1, 1, j0 + 3, j0)

                @pl.when(j0 + 2 < NBLK)
                def _():
                    stage(j0 + 2, 2, j0 + 4, j0)

                return carry

            lax.fori_loop(0, (NBLK + 2) // 3, body, 0)
            swait(NBLK - 1)
            plsc.subcore_barrier()

            @pl.when(s < 15)
            def _():
                pltpu.sync_copy(acc.at[pl.ds(s * ROWS_A, ROWS_A)],
                                out.at[pl.ds(base + s * ROWS_A, ROWS_A)])

            @pl.when(s == 15)
            def _():
                pltpu.sync_copy(acc.at[pl.ds(15 * ROWS_A, ROWS_B)],
                                out.at[pl.ds(base + 15 * ROWS_A, ROWS_B)])

            plsc.subcore_barrier()

    return k


def _sc_deg():
    mesh = plsc.VectorSubcoreMesh(core_axis_name="c", subcore_axis_name="s")

    @functools.partial(
        pl.kernel,
        out_type=jax.ShapeDtypeStruct((NP, 16), jnp.float32),
        mesh=mesh,
        scratch_types=[
            pltpu.VMEM((NBLK, KB), jnp.int32),
            pltpu.VMEM((KB, 16), jnp.float32),
            pltpu.VMEM((ROWS_A, 16), jnp.float32),
            pltpu.VMEM_SHARED((NP, 16), jnp.float32),
        ],
        compiler_params=pltpu.CompilerParams(use_tc_tiling_on_sc=False),
    )
    def k(dst, out, dst_v, ones_v, rows_v, acc):
        c = lax.axis_index("c")
        s = lax.axis_index("s")

        @pl.when(c == 0)
        def _():
            pltpu.sync_copy(dst.at[s], dst_v)

            def fill_ones(i, carry):
                ones_v[i, :] = jnp.full((16,), 1.0, jnp.float32)
                return carry

            lax.fori_loop(0, KB, fill_ones, 0)

            def fill_rows(i, carry):
                rows_v[i, :] = jnp.full((16,), 1.0, jnp.float32)
                return carry

            lax.fori_loop(0, ROWS_A, fill_rows, 0)
            @pl.when(s < 15)
            def _():
                pltpu.sync_copy(rows_v, acc.at[pl.ds(s * ROWS_A, ROWS_A)])

            @pl.when(s == 15)
            def _():
                pltpu.sync_copy(rows_v.at[pl.ds(0, ROWS_B)],
                                acc.at[pl.ds(15 * ROWS_A, ROWS_B)])

            plsc.subcore_barrier()

            def body(j, carry):
                pltpu.sync_copy(ones_v, acc.at[dst_v.at[j]], add=True)
                return carry

            lax.fori_loop(0, NBLK, body, 0)
            plsc.subcore_barrier()

            @pl.when(s < 15)
            def _():
                pltpu.sync_copy(acc.at[pl.ds(s * ROWS_A, ROWS_A)],
                                out.at[pl.ds(s * ROWS_A, ROWS_A)])

            @pl.when(s == 15)
            def _():
                pltpu.sync_copy(acc.at[pl.ds(15 * ROWS_A, ROWS_B)],
                                out.at[pl.ds(15 * ROWS_A, ROWS_B)])

    return k


def _leaky(x):
    return jnp.where(x > 0, x, 0.01 * x)


def _tc_matmul_first(x, w, deg):
    k_in, d_out = w.shape
    cn = d_out // 128

    def body(x_ref, w_ref, deg_ref, out_ref):
        y = jnp.dot(x_ref[...], w_ref[...],
                    preferred_element_type=jnp.float32)
        y = y * lax.rsqrt(deg_ref[:, 0:1])
        for c in range(cn):
            out_ref[c] = y[:, c * 128:(c + 1) * 128]

    return pl.pallas_call(
        body,
        grid=(N // BM,),
        in_specs=[
            pl.BlockSpec((BM, k_in), lambda m: (m, 0)),
            pl.BlockSpec((k_in, d_out), lambda m: (0, 0)),
            pl.BlockSpec((BM, 16), lambda m: (m, 0)),
        ],
        out_specs=pl.BlockSpec((cn, BM, 128), lambda m: (0, m, 0)),
        out_shape=jax.ShapeDtypeStruct((cn, NP, 128), jnp.float32),
    )(x, w, deg)


def _tc_matmul(agg, w, b_prev, deg):
    k_in, d_out = w.shape
    ck, cn = k_in // 128, d_out // 128

    def body(agg_ref, w_ref, b_ref, deg_ref, out_ref):
        dinv = lax.rsqrt(deg_ref[:, 0:1])
        acc = jnp.zeros((BM, d_out), jnp.float32)
        for c in range(ck):
            h = _leaky(dinv * agg_ref[c] + b_ref[0, c * 128:(c + 1) * 128])
            acc += jnp.dot(h, w_ref[c * 128:(c + 1) * 128, :],
                           preferred_element_type=jnp.float32)
        acc = acc * dinv
        for c in range(cn):
            out_ref[c] = acc[:, c * 128:(c + 1) * 128]

    return pl.pallas_call(
        body,
        grid=(N // BM,),
        in_specs=[
            pl.BlockSpec((ck, BM, 128), lambda m: (0, m, 0)),
            pl.BlockSpec((k_in, d_out), lambda m: (0, 0)),
            pl.BlockSpec((1, k_in), lambda m: (0, 0)),
            pl.BlockSpec((BM, 16), lambda m: (m, 0)),
        ],
        out_specs=pl.BlockSpec((cn, BM, 128), lambda m: (0, m, 0)),
        out_shape=jax.ShapeDtypeStruct((cn, NP, 128), jnp.float32),
    )(agg, w, b_prev.reshape(1, k_in), deg)


def _tc_epilogue(agg, b, deg, d_out):
    cn = d_out // 128
    grid = (N // BM, cn)

    def body(agg_ref, b_ref, deg_ref, out_ref):
        out_ref[...] = lax.rsqrt(deg_ref[:, 0:1]) * agg_ref[0] + b_ref[0, 0]

    return pl.pallas_call(
        body,
        grid=grid,
        in_specs=[
            pl.BlockSpec((1, BM, 128), lambda m, n: (n, m, 0)),
            pl.BlockSpec((1, 1, 128), lambda m, n: (n, 0, 0)),
            pl.BlockSpec((BM, 16), lambda m, n: (m, 0)),
        ],
        out_specs=pl.BlockSpec((BM, 128), lambda m, n: (m, n)),
        out_shape=jax.ShapeDtypeStruct((N, d_out), jnp.float32),
    )(agg, b.reshape(cn, 1, 128), deg)


_deg_kernel = _sc_deg()
_agg4 = _sc_agg(4)
_agg2 = _sc_agg(2)


def kernel(x, edge_index, W_enc, b_enc, W_h0, b_h0, W_h1, b_h1, W_h2, b_h2,
           W_dec, b_dec):
    src = edge_index[0].reshape(NS, NBLK, KB)
    dst = edge_index[1].reshape(NS, NBLK, KB)

    deg = _deg_kernel(dst)

    y = _tc_matmul_first(x, W_enc, deg)
    agg = _agg4(y.reshape(4 * NP, 128), src, dst)
    y = _tc_matmul(agg.reshape(4, NP, 128), W_h0, b_enc, deg)
    agg = _agg4(y.reshape(4 * NP, 128), src, dst)
    y = _tc_matmul(agg.reshape(4, NP, 128), W_h1, b_h0, deg)
    agg = _agg4(y.reshape(4 * NP, 128), src, dst)
    y = _tc_matmul(agg.reshape(4, NP, 128), W_h2, b_h1, deg)
    agg = _agg4(y.reshape(4 * NP, 128), src, dst)
    y = _tc_matmul(agg.reshape(4, NP, 128), W_dec, b_h2, deg)
    agg = _agg2(y.reshape(2 * NP, 128), src, dst)
    return _tc_epilogue(agg.reshape(2, NP, 128), b_dec, deg, 256)


# speedup vs baseline: 3.2542x; 1.1998x over previous
import functools

import jax
import jax.numpy as jnp
from jax import lax
from jax.experimental import pallas as pl
from jax.experimental.pallas import tpu as pltpu
from jax.experimental.pallas import tpu_sc as plsc

N = 10000
NP = 10008
E = 160000
NS = 16
NC = 2
EPT = E // NS
KB = 80
NBLK = 125
ROWS_A = 632
ROWS_B = NP - 15 * ROWS_A
BM = 1000


def _sc_agg(width, dtype):
    mesh = plsc.VectorSubcoreMesh(core_axis_name="c", subcore_axis_name="s")

    @functools.partial(
        pl.kernel,
        out_type=jax.ShapeDtypeStruct((NC * NP, width), dtype),
        mesh=mesh,
        scratch_types=[
            pltpu.VMEM((NBLK, KB), jnp.int32),
            pltpu.VMEM((NBLK, KB), jnp.int32),
            pltpu.VMEM((KB, width), dtype),
            pltpu.VMEM((KB, width), dtype),
            pltpu.VMEM((KB, width), dtype),
            pltpu.VMEM_SHARED((NP, width), dtype),
            pltpu.SemaphoreType.DMA,
            pltpu.SemaphoreType.DMA,
            pltpu.SemaphoreType.DMA,
            pltpu.SemaphoreType.DMA,
            pltpu.SemaphoreType.DMA,
            pltpu.SemaphoreType.DMA,
        ],
        compiler_params=pltpu.CompilerParams(use_tc_tiling_on_sc=False),
    )
    def k(y, ei, out, src_v, dst_v, buf_a, buf_b, buf_c, acc,
          sem_a, sem_b, sem_c, sem_sa, sem_sb, sem_sc):
        c = lax.axis_index("c")
        s = lax.axis_index("s")
        pltpu.sync_copy(ei.at[s], src_v)
        pltpu.sync_copy(ei.at[NS + s], dst_v)
        bufs = (buf_a, buf_b, buf_c)
        gsems = (sem_a, sem_b, sem_c)
        ssems = (sem_sa, sem_sb, sem_sc)

        def gwait(r):
            pltpu.make_async_copy(
                y.at[src_v.at[0]], bufs[r % 3], gsems[r % 3]).wait()

        def swait(r):
            pltpu.make_async_copy(
                bufs[r % 3], acc.at[dst_v.at[0]], ssems[r % 3]).wait()

        base = c * NP
        ytab = y.at[pl.ds(base, NP)]
        pltpu.async_copy(ytab.at[src_v.at[0]], buf_a, sem_a)
        pltpu.async_copy(ytab.at[src_v.at[1]], buf_b, sem_b)

        @pl.when(s < 15)
        def _():
            pltpu.sync_copy(y.at[pl.ds(base + s * ROWS_A, ROWS_A)],
                            acc.at[pl.ds(s * ROWS_A, ROWS_A)])

        @pl.when(s == 15)
        def _():
            pltpu.sync_copy(y.at[pl.ds(base + 15 * ROWS_A, ROWS_B)],
                            acc.at[pl.ds(15 * ROWS_A, ROWS_B)])

        plsc.subcore_barrier()

        def stage(j, r, fire_j, loop_j0):
            gwait(r)
            pltpu.async_copy(bufs[r % 3], acc.at[dst_v.at[j]],
                             ssems[r % 3], add=True)

            @pl.when(loop_j0 + r >= 1)
            def _():
                swait(r + 2)

            @pl.when(fire_j < NBLK)
            def _():
                pltpu.async_copy(ytab.at[src_v.at[fire_j]],
                                 bufs[(r + 2) % 3], gsems[(r + 2) % 3])

        def body(i, carry):
            j0 = 3 * i
            stage(j0, 0, j0 + 2, j0)

            @pl.when(j0 + 1 < NBLK)
            def _():
                stage(j0 + 1, 1, j0 + 3, j0)

            @pl.when(j0 + 2 < NBLK)
            def _():
                stage(j0 + 2, 2, j0 + 4, j0)

            return carry

        lax.fori_loop(0, (NBLK + 2) // 3, body, 0)
        swait(NBLK - 1)
        plsc.subcore_barrier()

        @pl.when(s < 15)
        def _():
            pltpu.sync_copy(acc.at[pl.ds(s * ROWS_A, ROWS_A)],
                            out.at[pl.ds(base + s * ROWS_A, ROWS_A)])

        @pl.when(s == 15)
        def _():
            pltpu.sync_copy(acc.at[pl.ds(15 * ROWS_A, ROWS_B)],
                            out.at[pl.ds(base + 15 * ROWS_A, ROWS_B)])

    return k


def _sc_deg():
    mesh = plsc.VectorSubcoreMesh(core_axis_name="c", subcore_axis_name="s")

    @functools.partial(
        pl.kernel,
        out_type=jax.ShapeDtypeStruct((NP, 16), jnp.float32),
        mesh=mesh,
        scratch_types=[
            pltpu.VMEM((NBLK, KB), jnp.int32),
            pltpu.VMEM((KB, 16), jnp.float32),
            pltpu.VMEM((ROWS_A, 16), jnp.float32),
            pltpu.VMEM_SHARED((NP, 16), jnp.float32),
        ],
        compiler_params=pltpu.CompilerParams(use_tc_tiling_on_sc=False),
    )
    def k(ei, out, dst_v, ones_v, rows_v, acc):
        c = lax.axis_index("c")
        s = lax.axis_index("s")

        @pl.when(c == 0)
        def _():
            pltpu.sync_copy(ei.at[NS + s], dst_v)

            def fill_ones(i, carry):
                ones_v[i, :] = jnp.full((16,), 1.0, jnp.float32)
                return carry

            lax.fori_loop(0, KB, fill_ones, 0)

            def fill_rows(i, carry):
                rows_v[i, :] = jnp.full((16,), 1.0, jnp.float32)
                return carry

            lax.fori_loop(0, ROWS_A, fill_rows, 0)

            @pl.when(s < 15)
            def _():
                pltpu.sync_copy(rows_v, acc.at[pl.ds(s * ROWS_A, ROWS_A)])

            @pl.when(s == 15)
            def _():
                pltpu.sync_copy(rows_v.at[pl.ds(0, ROWS_B)],
                                acc.at[pl.ds(15 * ROWS_A, ROWS_B)])

            plsc.subcore_barrier()

            def body(j, carry):
                pltpu.sync_copy(ones_v, acc.at[dst_v.at[j]], add=True)
                return carry

            lax.fori_loop(0, NBLK, body, 0)
            plsc.subcore_barrier()

            @pl.when(s < 15)
            def _():
                pltpu.sync_copy(acc.at[pl.ds(s * ROWS_A, ROWS_A)],
                                out.at[pl.ds(s * ROWS_A, ROWS_A)])

            @pl.when(s == 15)
            def _():
                pltpu.sync_copy(acc.at[pl.ds(15 * ROWS_A, ROWS_B)],
                                out.at[pl.ds(15 * ROWS_A, ROWS_B)])

    return k


def _leaky(x):
    return jnp.where(x > 0, x, 0.01 * x)


def _tc_matmul_first(x, w, deg):
    k_in, d_out = w.shape
    wc = d_out // NC

    def body(x_ref, w_ref, deg_ref, out_ref):
        y = jnp.dot(x_ref[...], w_ref[...],
                    preferred_element_type=jnp.float32)
        y = (y * lax.rsqrt(deg_ref[:, 0:1])).astype(jnp.bfloat16)
        for cc in range(NC):
            out_ref[cc] = y[:, cc * wc:(cc + 1) * wc]

    return pl.pallas_call(
        body,
        grid=(N // BM,),
        in_specs=[
            pl.BlockSpec((BM, k_in), lambda m: (m, 0)),
            pl.BlockSpec((k_in, d_out), lambda m: (0, 0)),
            pl.BlockSpec((BM, 16), lambda m: (m, 0)),
        ],
        out_specs=pl.BlockSpec((NC, BM, wc), lambda m: (0, m, 0)),
        out_shape=jax.ShapeDtypeStruct((NC, NP, wc), jnp.bfloat16),
    )(x, w, deg)


def _tc_matmul(agg, w, b_prev, deg, out_dtype, out_wc):
    k_in, d_out = w.shape
    wc = k_in // NC
    cn = d_out // out_wc

    def body(agg_ref, w_ref, b_ref, deg_ref, out_ref):
        dinv = lax.rsqrt(deg_ref[:, 0:1])
        acc = jnp.zeros((BM, d_out), jnp.float32)
        for cc in range(NC):
            h = _leaky(dinv * agg_ref[cc].astype(jnp.float32)
                       + b_ref[0, cc * wc:(cc + 1) * wc])
            acc += jnp.dot(h, w_ref[cc * wc:(cc + 1) * wc, :],
                           preferred_element_type=jnp.float32)
        acc = (acc * dinv).astype(out_dtype)
        for cc in range(cn):
            out_ref[cc] = acc[:, cc * out_wc:(cc + 1) * out_wc]

    return pl.pallas_call(
        body,
        grid=(N // BM,),
        in_specs=[
            pl.BlockSpec((NC, BM, wc), lambda m: (0, m, 0)),
            pl.BlockSpec((k_in, d_out), lambda m: (0, 0)),
            pl.BlockSpec((1, k_in), lambda m: (0, 0)),
            pl.BlockSpec((BM, 16), lambda m: (m, 0)),
        ],
        out_specs=pl.BlockSpec((cn, BM, out_wc), lambda m: (0, m, 0)),
        out_shape=jax.ShapeDtypeStruct((cn, NP, out_wc), out_dtype),
    )(agg, w, b_prev.reshape(1, k_in), deg)


def _tc_epilogue(agg, b, deg, d_out):
    cn = d_out // 128
    grid = (N // BM, cn)

    def body(agg_ref, b_ref, deg_ref, out_ref):
        out_ref[...] = lax.rsqrt(deg_ref[:, 0:1]) * agg_ref[0] + b_ref[0, 0]

    return pl.pallas_call(
        body,
        grid=grid,
        in_specs=[
            pl.BlockSpec((1, BM, 128), lambda m, n: (n, m, 0)),
            pl.BlockSpec((1, 1, 128), lambda m, n: (n, 0, 0)),
            pl.BlockSpec((BM, 16), lambda m, n: (m, 0)),
        ],
        out_specs=pl.BlockSpec((BM, 128), lambda m, n: (m, n)),
        out_shape=jax.ShapeDtypeStruct((N, d_out), jnp.float32),
    )(agg, b.reshape(cn, 1, 128), deg)


_deg_kernel = _sc_deg()
_agg_bf = _sc_agg(256, jnp.bfloat16)
_agg_f32 = _sc_agg(128, jnp.float32)


def kernel(x, edge_index, W_enc, b_enc, W_h0, b_h0, W_h1, b_h1, W_h2, b_h2,
           W_dec, b_dec):
    ei = edge_index.reshape(2 * NS, NBLK, KB)

    deg = _deg_kernel(ei)

    y = _tc_matmul_first(x, W_enc, deg)
    agg = _agg_bf(y.reshape(2 * NP, 256), ei)
    y = _tc_matmul(agg.reshape(2, NP, 256), W_h0, b_enc, deg, jnp.bfloat16, 256)
    agg = _agg_bf(y.reshape(2 * NP, 256), ei)
    y = _tc_matmul(agg.reshape(2, NP, 256), W_h1, b_h0, deg, jnp.bfloat16, 256)
    agg = _agg_bf(y.reshape(2 * NP, 256), ei)
    y = _tc_matmul(agg.reshape(2, NP, 256), W_h2, b_h1, deg, jnp.bfloat16, 256)
    agg = _agg_bf(y.reshape(2 * NP, 256), ei)
    y = _tc_matmul(agg.reshape(2, NP, 256), W_dec, b_h2, deg, jnp.float32, 128)
    agg = _agg_f32(y.reshape(2 * NP, 128), ei)
    return _tc_epilogue(agg.reshape(2, NP, 128), b_dec, deg, 256)


# speedup vs baseline: 3.2557x; 1.0005x over previous
import functools

import jax
import jax.numpy as jnp
from jax import lax
from jax.experimental import pallas as pl
from jax.experimental.pallas import tpu as pltpu
from jax.experimental.pallas import tpu_sc as plsc

N = 10000
NP = 10008
E = 160000
NS = 16
NC = 2
EPT = E // NS
KB = 80
NBLK = 125
ROWS_A = 632
ROWS_B = NP - 15 * ROWS_A
BM = 1000


def _sc_agg(width, dtype):
    mesh = plsc.VectorSubcoreMesh(core_axis_name="c", subcore_axis_name="s")

    @functools.partial(
        pl.kernel,
        out_type=jax.ShapeDtypeStruct((NC, NP, width), dtype),
        mesh=mesh,
        scratch_types=[
            pltpu.VMEM((NBLK, KB), jnp.int32),
            pltpu.VMEM((NBLK, KB), jnp.int32),
            pltpu.VMEM((KB, width), dtype),
            pltpu.VMEM((KB, width), dtype),
            pltpu.VMEM((KB, width), dtype),
            pltpu.VMEM_SHARED((NP, width), dtype),
            pltpu.SemaphoreType.DMA,
            pltpu.SemaphoreType.DMA,
            pltpu.SemaphoreType.DMA,
            pltpu.SemaphoreType.DMA,
            pltpu.SemaphoreType.DMA,
            pltpu.SemaphoreType.DMA,
        ],
        compiler_params=pltpu.CompilerParams(use_tc_tiling_on_sc=False),
    )
    def k(y, ei, out, src_v, dst_v, buf_a, buf_b, buf_c, acc,
          sem_a, sem_b, sem_c, sem_sa, sem_sb, sem_sc):
        c = lax.axis_index("c")
        s = lax.axis_index("s")
        pltpu.sync_copy(ei.at[s], src_v)
        pltpu.sync_copy(ei.at[NS + s], dst_v)
        bufs = (buf_a, buf_b, buf_c)
        gsems = (sem_a, sem_b, sem_c)
        ssems = (sem_sa, sem_sb, sem_sc)

        def gwait(r):
            pltpu.make_async_copy(
                ytab.at[src_v.at[0]], bufs[r % 3], gsems[r % 3]).wait()

        def swait(r):
            pltpu.make_async_copy(
                bufs[r % 3], acc.at[dst_v.at[0]], ssems[r % 3]).wait()

        ytab = y.at[c]
        otab = out.at[c]
        pltpu.async_copy(ytab.at[src_v.at[0]], buf_a, sem_a)
        pltpu.async_copy(ytab.at[src_v.at[1]], buf_b, sem_b)

        @pl.when(s < 15)
        def _():
            pltpu.sync_copy(ytab.at[pl.ds(s * ROWS_A, ROWS_A)],
                            acc.at[pl.ds(s * ROWS_A, ROWS_A)])

        @pl.when(s == 15)
        def _():
            pltpu.sync_copy(ytab.at[pl.ds(15 * ROWS_A, ROWS_B)],
                            acc.at[pl.ds(15 * ROWS_A, ROWS_B)])

        plsc.subcore_barrier()

        def stage(j, r, fire_j, loop_j0):
            gwait(r)
            pltpu.async_copy(bufs[r % 3], acc.at[dst_v.at[j]],
                             ssems[r % 3], add=True)

            @pl.when(loop_j0 + r >= 1)
            def _():
                swait(r + 2)

            @pl.when(fire_j < NBLK)
            def _():
                pltpu.async_copy(ytab.at[src_v.at[fire_j]],
                                 bufs[(r + 2) % 3], gsems[(r + 2) % 3])

        def body(i, carry):
            j0 = 3 * i
            stage(j0, 0, j0 + 2, j0)

            @pl.when(j0 + 1 < NBLK)
            def _():
                stage(j0 + 1, 1, j0 + 3, j0)

            @pl.when(j0 + 2 < NBLK)
            def _():
                stage(j0 + 2, 2, j0 + 4, j0)

            return carry

        lax.fori_loop(0, (NBLK + 2) // 3, body, 0)
        swait(NBLK - 1)
        plsc.subcore_barrier()

        @pl.when(s < 15)
        def _():
            pltpu.sync_copy(acc.at[pl.ds(s * ROWS_A, ROWS_A)],
                            otab.at[pl.ds(s * ROWS_A, ROWS_A)])

        @pl.when(s == 15)
        def _():
            pltpu.sync_copy(acc.at[pl.ds(15 * ROWS_A, ROWS_B)],
                            otab.at[pl.ds(15 * ROWS_A, ROWS_B)])

    return k


def _sc_deg():
    mesh = plsc.VectorSubcoreMesh(core_axis_name="c", subcore_axis_name="s")

    @functools.partial(
        pl.kernel,
        out_type=jax.ShapeDtypeStruct((NP, 16), jnp.float32),
        mesh=mesh,
        scratch_types=[
            pltpu.VMEM((NBLK, KB), jnp.int32),
            pltpu.VMEM((KB, 16), jnp.float32),
            pltpu.VMEM((ROWS_A, 16), jnp.float32),
            pltpu.VMEM_SHARED((NP, 16), jnp.float32),
        ],
        compiler_params=pltpu.CompilerParams(use_tc_tiling_on_sc=False),
    )
    def k(ei, out, dst_v, ones_v, rows_v, acc):
        c = lax.axis_index("c")
        s = lax.axis_index("s")

        @pl.when(c == 0)
        def _():
            pltpu.sync_copy(ei.at[NS + s], dst_v)

            def fill_ones(i, carry):
                ones_v[i, :] = jnp.full((16,), 1.0, jnp.float32)
                return carry

            lax.fori_loop(0, KB, fill_ones, 0)

            def fill_rows(i, carry):
                rows_v[i, :] = jnp.full((16,), 1.0, jnp.float32)
                return carry

            lax.fori_loop(0, ROWS_A, fill_rows, 0)

            @pl.when(s < 15)
            def _():
                pltpu.sync_copy(rows_v, acc.at[pl.ds(s * ROWS_A, ROWS_A)])

            @pl.when(s == 15)
            def _():
                pltpu.sync_copy(rows_v.at[pl.ds(0, ROWS_B)],
                                acc.at[pl.ds(15 * ROWS_A, ROWS_B)])

            plsc.subcore_barrier()

            def body(j, carry):
                pltpu.sync_copy(ones_v, acc.at[dst_v.at[j]], add=True)
                return carry

            lax.fori_loop(0, NBLK, body, 0)
            plsc.subcore_barrier()

            @pl.when(s < 15)
            def _():
                pltpu.sync_copy(acc.at[pl.ds(s * ROWS_A, ROWS_A)],
                                out.at[pl.ds(s * ROWS_A, ROWS_A)])

            @pl.when(s == 15)
            def _():
                pltpu.sync_copy(acc.at[pl.ds(15 * ROWS_A, ROWS_B)],
                                out.at[pl.ds(15 * ROWS_A, ROWS_B)])

    return k


def _leaky(x):
    return jnp.where(x > 0, x, 0.01 * x)


def _tc_matmul_first(x, w, deg):
    k_in, d_out = w.shape
    wc = d_out // NC

    def body(x_ref, w_ref, deg_ref, out_ref):
        y = jnp.dot(x_ref[...], w_ref[...],
                    preferred_element_type=jnp.float32)
        y = (y * lax.rsqrt(deg_ref[:, 0:1])).astype(jnp.bfloat16)
        for cc in range(NC):
            out_ref[cc] = y[:, cc * wc:(cc + 1) * wc]

    return pl.pallas_call(
        body,
        grid=(N // BM,),
        in_specs=[
            pl.BlockSpec((BM, k_in), lambda m: (m, 0)),
            pl.BlockSpec((k_in, d_out), lambda m: (0, 0)),
            pl.BlockSpec((BM, 16), lambda m: (m, 0)),
        ],
        out_specs=pl.BlockSpec((NC, BM, wc), lambda m: (0, m, 0)),
        out_shape=jax.ShapeDtypeStruct((NC, NP, wc), jnp.bfloat16),
    )(x, w, deg)


def _tc_matmul(agg, w, b_prev, deg, out_dtype, out_wc):
    k_in, d_out = w.shape
    wc = k_in // NC
    cn = d_out // out_wc

    def body(agg_ref, w_ref, b_ref, deg_ref, out_ref):
        dinv = lax.rsqrt(deg_ref[:, 0:1])
        acc = jnp.zeros((BM, d_out), jnp.float32)
        for cc in range(NC):
            h = _leaky(dinv * agg_ref[cc].astype(jnp.float32)
                       + b_ref[0, cc * wc:(cc + 1) * wc])
            acc += jnp.dot(h, w_ref[cc * wc:(cc + 1) * wc, :],
                           preferred_element_type=jnp.float32)
        acc = (acc * dinv).astype(out_dtype)
        for cc in range(cn):
            out_ref[cc] = acc[:, cc * out_wc:(cc + 1) * out_wc]

    return pl.pallas_call(
        body,
        grid=(N // BM,),
        in_specs=[
            pl.BlockSpec((NC, BM, wc), lambda m: (0, m, 0)),
            pl.BlockSpec((k_in, d_out), lambda m: (0, 0)),
            pl.BlockSpec((1, k_in), lambda m: (0, 0)),
            pl.BlockSpec((BM, 16), lambda m: (m, 0)),
        ],
        out_specs=pl.BlockSpec((cn, BM, out_wc), lambda m: (0, m, 0)),
        out_shape=jax.ShapeDtypeStruct((cn, NP, out_wc), out_dtype),
    )(agg, w, b_prev.reshape(1, k_in), deg)


def _tc_epilogue(agg, b, deg, d_out):
    cn = d_out // 128
    grid = (N // BM, cn)

    def body(agg_ref, b_ref, deg_ref, out_ref):
        out_ref[...] = lax.rsqrt(deg_ref[:, 0:1]) * agg_ref[0] + b_ref[0, 0]

    return pl.pallas_call(
        body,
        grid=grid,
        in_specs=[
            pl.BlockSpec((1, BM, 128), lambda m, n: (n, m, 0)),
            pl.BlockSpec((1, 1, 128), lambda m, n: (n, 0, 0)),
            pl.BlockSpec((BM, 16), lambda m, n: (m, 0)),
        ],
        out_specs=pl.BlockSpec((BM, 128), lambda m, n: (m, n)),
        out_shape=jax.ShapeDtypeStruct((N, d_out), jnp.float32),
    )(agg, b.reshape(cn, 1, 128), deg)


_deg_kernel = _sc_deg()
_agg_bf = _sc_agg(256, jnp.bfloat16)
_agg_f32 = _sc_agg(128, jnp.float32)


def kernel(x, edge_index, W_enc, b_enc, W_h0, b_h0, W_h1, b_h1, W_h2, b_h2,
           W_dec, b_dec):
    ei = edge_index.reshape(2 * NS, NBLK, KB)

    deg = _deg_kernel(ei)

    y = _tc_matmul_first(x, W_enc, deg)
    agg = _agg_bf(y, ei)
    y = _tc_matmul(agg, W_h0, b_enc, deg, jnp.bfloat16, 256)
    agg = _agg_bf(y, ei)
    y = _tc_matmul(agg, W_h1, b_h0, deg, jnp.bfloat16, 256)
    agg = _agg_bf(y, ei)
    y = _tc_matmul(agg, W_h2, b_h1, deg, jnp.bfloat16, 256)
    agg = _agg_bf(y, ei)
    y = _tc_matmul(agg, W_dec, b_h2, deg, jnp.float32, 128)
    agg = _agg_f32(y, ei)
    return _tc_epilogue(agg, b_dec, deg, 256)


# speedup vs baseline: 3.2827x; 1.0083x over previous
import functools

import jax
import jax.numpy as jnp
from jax import lax
from jax.experimental import pallas as pl
from jax.experimental.pallas import tpu as pltpu
from jax.experimental.pallas import tpu_sc as plsc

N = 10000
NP = 10008
E = 160000
NS = 16
NC = 2
EPT = E // NS
KB = 80
NBLK = 125
ROWS_A = 632
ROWS_B = NP - 15 * ROWS_A
BM = 1000


def _sc_agg(width, dtype):
    mesh = plsc.VectorSubcoreMesh(core_axis_name="c", subcore_axis_name="s")

    @functools.partial(
        pl.kernel,
        out_type=jax.ShapeDtypeStruct((NC, NP, width), dtype),
        mesh=mesh,
        scratch_types=[
            pltpu.VMEM((NBLK, KB), jnp.int32),
            pltpu.VMEM((NBLK, KB), jnp.int32),
            pltpu.VMEM((KB, width), dtype),
            pltpu.VMEM((KB, width), dtype),
            pltpu.VMEM((KB, width), dtype),
            pltpu.VMEM_SHARED((NP, width), dtype),
            pltpu.SemaphoreType.DMA,
            pltpu.SemaphoreType.DMA,
            pltpu.SemaphoreType.DMA,
            pltpu.SemaphoreType.DMA,
            pltpu.SemaphoreType.DMA,
            pltpu.SemaphoreType.DMA,
        ],
        compiler_params=pltpu.CompilerParams(use_tc_tiling_on_sc=False),
    )
    def k(y, ei, out, src_v, dst_v, buf_a, buf_b, buf_c, acc,
          sem_a, sem_b, sem_c, sem_sa, sem_sb, sem_sc):
        c = lax.axis_index("c")
        s = lax.axis_index("s")
        pltpu.sync_copy(ei.at[s], src_v)
        pltpu.sync_copy(ei.at[NS + s], dst_v)
        bufs = (buf_a, buf_b, buf_c)
        gsems = (sem_a, sem_b, sem_c)
        ssems = (sem_sa, sem_sb, sem_sc)

        def gwait(r):
            pltpu.make_async_copy(
                ytab.at[src_v.at[0]], bufs[r % 3], gsems[r % 3]).wait()

        def swait(r):
            pltpu.make_async_copy(
                bufs[r % 3], acc.at[dst_v.at[0]], ssems[r % 3]).wait()

        ytab = y.at[c]
        otab = out.at[c]
        pltpu.async_copy(ytab.at[src_v.at[0]], buf_a, sem_a)
        pltpu.async_copy(ytab.at[src_v.at[1]], buf_b, sem_b)

        @pl.when(s < 15)
        def _():
            pltpu.sync_copy(ytab.at[pl.ds(s * ROWS_A, ROWS_A)],
                            acc.at[pl.ds(s * ROWS_A, ROWS_A)])

        @pl.when(s == 15)
        def _():
            pltpu.sync_copy(ytab.at[pl.ds(15 * ROWS_A, ROWS_B)],
                            acc.at[pl.ds(15 * ROWS_A, ROWS_B)])

        plsc.subcore_barrier()

        def stage(j, r, fire_j, loop_j0):
            gwait(r)
            pltpu.async_copy(bufs[r % 3], acc.at[dst_v.at[j]],
                             ssems[r % 3], add=True)

            @pl.when(loop_j0 + r >= 1)
            def _():
                swait(r + 2)

            @pl.when(fire_j < NBLK)
            def _():
                pltpu.async_copy(ytab.at[src_v.at[fire_j]],
                                 bufs[(r + 2) % 3], gsems[(r + 2) % 3])

        def body(i, carry):
            j0 = 3 * i
            stage(j0, 0, j0 + 2, j0)

            @pl.when(j0 + 1 < NBLK)
            def _():
                stage(j0 + 1, 1, j0 + 3, j0)

            @pl.when(j0 + 2 < NBLK)
            def _():
                stage(j0 + 2, 2, j0 + 4, j0)

            return carry

        lax.fori_loop(0, (NBLK + 2) // 3, body, 0)
        swait(NBLK - 1)
        plsc.subcore_barrier()

        @pl.when(s < 15)
        def _():
            pltpu.sync_copy(acc.at[pl.ds(s * ROWS_A, ROWS_A)],
                            otab.at[pl.ds(s * ROWS_A, ROWS_A)])

        @pl.when(s == 15)
        def _():
            pltpu.sync_copy(acc.at[pl.ds(15 * ROWS_A, ROWS_B)],
                            otab.at[pl.ds(15 * ROWS_A, ROWS_B)])

    return k


def _sc_deg():
    mesh = plsc.VectorSubcoreMesh(core_axis_name="c", subcore_axis_name="s")

    @functools.partial(
        pl.kernel,
        out_type=jax.ShapeDtypeStruct((NP, 16), jnp.float32),
        mesh=mesh,
        scratch_types=[
            pltpu.VMEM((NBLK, KB), jnp.int32),
            pltpu.VMEM((KB, 16), jnp.float32),
            pltpu.VMEM((ROWS_A, 16), jnp.float32),
            pltpu.VMEM_SHARED((NP, 16), jnp.float32),
        ],
        compiler_params=pltpu.CompilerParams(use_tc_tiling_on_sc=False),
    )
    def k(ei, out, dst_v, ones_v, rows_v, acc):
        c = lax.axis_index("c")
        s = lax.axis_index("s")

        @pl.when(c == 0)
        def _():
            pltpu.sync_copy(ei.at[NS + s], dst_v)

            def fill_ones(i, carry):
                ones_v[i, :] = jnp.full((16,), 1.0, jnp.float32)
                return carry

            lax.fori_loop(0, KB, fill_ones, 0)

            def fill_rows(i, carry):
                rows_v[i, :] = jnp.full((16,), 1.0, jnp.float32)
                return carry

            lax.fori_loop(0, ROWS_A, fill_rows, 0)

            @pl.when(s < 15)
            def _():
                pltpu.sync_copy(rows_v, acc.at[pl.ds(s * ROWS_A, ROWS_A)])

            @pl.when(s == 15)
            def _():
                pltpu.sync_copy(rows_v.at[pl.ds(0, ROWS_B)],
                                acc.at[pl.ds(15 * ROWS_A, ROWS_B)])

            plsc.subcore_barrier()

            def body(j, carry):
                pltpu.sync_copy(ones_v, acc.at[dst_v.at[j]], add=True)
                return carry

            lax.fori_loop(0, NBLK, body, 0)
            plsc.subcore_barrier()

            @pl.when(s < 15)
            def _():
                pltpu.sync_copy(acc.at[pl.ds(s * ROWS_A, ROWS_A)],
                                out.at[pl.ds(s * ROWS_A, ROWS_A)])

            @pl.when(s == 15)
            def _():
                pltpu.sync_copy(acc.at[pl.ds(15 * ROWS_A, ROWS_B)],
                                out.at[pl.ds(15 * ROWS_A, ROWS_B)])

    return k


def _leaky(x):
    return jnp.where(x > 0, x, 0.01 * x)


def _tc_matmul_first(x, w, deg):
    k_in, d_out = w.shape
    wc = d_out // NC

    def body(x_ref, w_ref, deg_ref, out_ref):
        y = jnp.dot(x_ref[...], w_ref[...],
                    preferred_element_type=jnp.float32)
        y = (y * lax.rsqrt(deg_ref[:, 0:1])).astype(jnp.bfloat16)
        for cc in range(NC):
            out_ref[cc] = y[:, cc * wc:(cc + 1) * wc]

    return pl.pallas_call(
        body,
        grid=(N // BM,),
        in_specs=[
            pl.BlockSpec((BM, k_in), lambda m: (m, 0)),
            pl.BlockSpec((k_in, d_out), lambda m: (0, 0)),
            pl.BlockSpec((BM, 16), lambda m: (m, 0)),
        ],
        out_specs=pl.BlockSpec((NC, BM, wc), lambda m: (0, m, 0)),
        out_shape=jax.ShapeDtypeStruct((NC, NP, wc), jnp.bfloat16),
    )(x, w, deg)


def _tc_matmul(agg, w, b_prev, deg, out_dtype, out_wc):
    k_in, d_out = w.shape
    wc = k_in // NC
    cn = d_out // out_wc

    def body(agg_ref, w_ref, b_ref, deg_ref, out_ref):
        dinv = lax.rsqrt(deg_ref[:, 0:1])
        acc = jnp.zeros((BM, d_out), jnp.float32)
        for cc in range(NC):
            h = _leaky(dinv * agg_ref[cc].astype(jnp.float32)
                       + b_ref[0, cc * wc:(cc + 1) * wc])
            acc += jnp.dot(h, w_ref[cc * wc:(cc + 1) * wc, :],
                           preferred_element_type=jnp.float32)
        acc = (acc * dinv).astype(out_dtype)
        for cc in range(cn):
            out_ref[cc] = acc[:, cc * out_wc:(cc + 1) * out_wc]

    return pl.pallas_call(
        body,
        grid=(N // BM,),
        in_specs=[
            pl.BlockSpec((NC, BM, wc), lambda m: (0, m, 0)),
            pl.BlockSpec((k_in, d_out), lambda m: (0, 0)),
            pl.BlockSpec((1, k_in), lambda m: (0, 0)),
            pl.BlockSpec((BM, 16), lambda m: (m, 0)),
        ],
        out_specs=pl.BlockSpec((cn, BM, out_wc), lambda m: (0, m, 0)),
        out_shape=jax.ShapeDtypeStruct((cn, NP, out_wc), out_dtype),
    )(agg, w, b_prev.reshape(1, k_in), deg)


def _tc_epilogue(agg, b, deg, d_out):
    wc = d_out // NC

    def body(agg_ref, b_ref, deg_ref, out_ref):
        dinv = lax.rsqrt(deg_ref[:, 0:1])
        for cc in range(NC):
            out_ref[:, cc * wc:(cc + 1) * wc] = (
                dinv * agg_ref[cc] + b_ref[cc, 0])

    return pl.pallas_call(
        body,
        grid=(N // BM,),
        in_specs=[
            pl.BlockSpec((NC, BM, wc), lambda m: (0, m, 0)),
            pl.BlockSpec((NC, 1, wc), lambda m: (0, 0, 0)),
            pl.BlockSpec((BM, 16), lambda m: (m, 0)),
        ],
        out_specs=pl.BlockSpec((BM, d_out), lambda m: (m, 0)),
        out_shape=jax.ShapeDtypeStruct((N, d_out), jnp.float32),
    )(agg, b.reshape(NC, 1, wc), deg)


_deg_kernel = _sc_deg()
_agg_bf = _sc_agg(256, jnp.bfloat16)
_agg_f32 = _sc_agg(128, jnp.float32)


def kernel(x, edge_index, W_enc, b_enc, W_h0, b_h0, W_h1, b_h1, W_h2, b_h2,
           W_dec, b_dec):
    ei = edge_index.reshape(2 * NS, NBLK, KB)

    deg = _deg_kernel(ei)

    y = _tc_matmul_first(x, W_enc, deg)
    agg = _agg_bf(y, ei)
    y = _tc_matmul(agg, W_h0, b_enc, deg, jnp.bfloat16, 256)
    agg = _agg_bf(y, ei)
    y = _tc_matmul(agg, W_h1, b_h0, deg, jnp.bfloat16, 256)
    agg = _agg_bf(y, ei)
    y = _tc_matmul(agg, W_h2, b_h1, deg, jnp.bfloat16, 256)
    agg = _agg_bf(y, ei)
    y = _tc_matmul(agg, W_dec, b_h2, deg, jnp.float32, 128)
    agg = _agg_f32(y, ei)
    return _tc_epilogue(agg, b_dec, deg, 256)
